# Initial kernel scaffold; baseline (speedup 1.0000x reference)
import jax
import jax.numpy as jnp
from jax.experimental import pallas as pl


def kernel(x, edge_weight, gc1_W, gc1_b, gc2_W, gc2_b, att_W, att_a, out_W, out_a, edge_index, idx_train, labels):
    raise NotImplementedError("write your pallas kernel here")


# speedup vs baseline: 6.3387x; 6.3387x over previous
import functools

import jax
import jax.numpy as jnp
from jax import lax
from jax.experimental import pallas as pl
from jax.experimental.pallas import tpu as pltpu
from jax.experimental.pallas import tpu_sc as plsc

N = 10000
NPAD = 10240
NFEAT = 128
NCLASS = 64
NHID1 = 128
NHID2 = 64
NHEADS = 8
ALPHA = 0.2
NTR = 2000
NTRP = 2048

E = 320000 + N
CK = 128
NW = 32
NCHUNK = 81
EPW = CK * NCHUNK
EPAD = EPW * NW
RPT = NPAD // 16
TRT = NTRP // 16

def _mesh():
    return plsc.VectorSubcoreMesh(core_axis_name="c", subcore_axis_name="s")


def _zeros16():
    return jnp.zeros((16,), jnp.float32)


def _lane_bcast(vec, idx16):
    dnums = lax.GatherDimensionNumbers(
        offset_dims=(), collapsed_slice_dims=(0,), start_index_map=(0,))
    return lax.gather(vec, idx16[:, None], dnums, (1,),
                      mode=lax.GatherScatterMode.PROMISE_IN_BOUNDS)


def _fill_zero(buf, nrows, ncols):
    def body(r, _):
        for d in range(ncols // 16):
            buf[r, pl.ds(d * 16, 16)] = _zeros16()
        return 0
    lax.fori_loop(0, nrows, body, 0)


def _tc0_body(x_ref, wbig_ref, asd_ref, lab_ref,
              xw1_ref, h0_ref, h1_ref, h2_ref, h3_ref, sd_ref, ss_ref, oh_ref):
    xb = x_ref[...]
    P = jnp.dot(xb, wbig_ref[...], preferred_element_type=jnp.float32)
    xw1_ref[...] = P[:, :NHID1]
    H = P[:, NHID1:]
    hrefs = (h0_ref, h1_ref, h2_ref, h3_ref)
    for p in range(4):
        hrefs[p][...] = H[:, p * 128:(p + 1) * 128]
    SDSS = jnp.dot(H, asd_ref[...], preferred_element_type=jnp.float32)
    sd_ref[...] = SDSS[:, :16]
    ss_ref[...] = SDSS[:, 16:]
    lab = lab_ref[...]
    io = lax.broadcasted_iota(jnp.int32, oh_ref.shape, 1)
    oh_ref[...] = jnp.where(io == lab, 1.0, 0.0).astype(jnp.float32)


def _tc0(x_p, wbig, asdss, lab_p):
    BR = 1280
    grid = NPAD // BR
    outs = [jax.ShapeDtypeStruct((NPAD, NHID1), jnp.float32)] \
        + [jax.ShapeDtypeStruct((NPAD, 128), jnp.float32)] * 4 \
        + [jax.ShapeDtypeStruct((NPAD, 16), jnp.float32)] * 2 \
        + [jax.ShapeDtypeStruct((NPAD, NCLASS), jnp.float32)]
    row = lambda i: (i, 0)
    return pl.pallas_call(
        _tc0_body,
        grid=(grid,),
        in_specs=[
            pl.BlockSpec((BR, NFEAT), row),
            pl.BlockSpec((NFEAT, 640), lambda i: (0, 0)),
            pl.BlockSpec((512, 32), lambda i: (0, 0)),
            pl.BlockSpec((BR, 1), row),
        ],
        out_specs=[
            pl.BlockSpec((BR, NHID1), row),
            pl.BlockSpec((BR, 128), row), pl.BlockSpec((BR, 128), row),
            pl.BlockSpec((BR, 128), row), pl.BlockSpec((BR, 128), row),
            pl.BlockSpec((BR, 16), row), pl.BlockSpec((BR, 16), row),
            pl.BlockSpec((BR, NCLASS), row),
        ],
        out_shape=outs,
    )(x_p, wbig, asdss, lab_p)


def _sca_body(xw1, h0, h1, h2, h3, sd, ss, ew, srci, dsti,
              g1p0, g1p1, hp00, hp01, hp10, hp11, hp20, hp21, hp30, hp31,
              rs0, rs1, e8s,
              sidx, didx, wv, rows, sdr, ssr, ev, zb, zb16, acc, rs, sem):
    c = lax.axis_index("c")
    s = lax.axis_index("s")
    wid = s * 2 + c
    base0 = wid * EPW
    r0 = s * RPT

    _fill_zero(zb, 64, 128)
    _fill_zero(zb16, 64, 16)

    def zero_acc():
        for i in range(RPT // 64):
            pltpu.sync_copy(zb, acc.at[pl.ds(r0 + i * 64, 64)])

    def zero_rs():
        for i in range(RPT // 64):
            pltpu.sync_copy(zb16, rs.at[pl.ds(r0 + i * 64, 64)])

    zero_acc()
    zero_rs()
    plsc.subcore_barrier()

    def spmm_chunk(ci, _):
        b = base0 + ci * CK
        pltpu.sync_copy(srci.at[pl.ds(b, CK)], sidx)
        pltpu.sync_copy(dsti.at[pl.ds(b, CK)], didx)
        pltpu.sync_copy(ew.at[pl.ds(b, CK)], wv)
        pltpu.async_copy(xw1.at[sidx], rows, sem).wait()

        def sc_group(g, _):
            wvec = wv[pl.ds(g * 16, 16)]
            for j in range(16):
                w = _lane_bcast(wvec, jnp.full((16,), j, jnp.int32))
                k = g * 16 + j
                for d in range(8):
                    rows[k, pl.ds(d * 16, 16)] = rows[k, pl.ds(d * 16, 16)] * w
            return 0
        lax.fori_loop(0, CK // 16, sc_group, 0)
        pltpu.sync_copy(rows, acc.at[didx], add=True)
        return 0
    lax.fori_loop(0, NCHUNK, spmm_chunk, 0)
    plsc.subcore_barrier()

    @pl.when(c == 0)
    def _():
        pltpu.sync_copy(acc.at[pl.ds(r0, RPT)], g1p0.at[pl.ds(r0, RPT)])

    @pl.when(c == 1)
    def _():
        pltpu.sync_copy(acc.at[pl.ds(r0, RPT)], g1p1.at[pl.ds(r0, RPT)])
    plsc.subcore_barrier()

    hlist = (h0, h1, h2, h3)
    hpout = ((hp00, hp01), (hp10, hp11), (hp20, hp21), (hp30, hp31))
    for p in range(4):
        zero_acc()
        plsc.subcore_barrier()
        i0 = jnp.full((16,), 2 * p, jnp.int32)
        i1 = jnp.full((16,), 2 * p + 1, jnp.int32)

        def gat_chunk(ci, _, _p=p, _h=hlist[p], _i0=i0, _i1=i1):
            b = base0 + ci * CK
            pltpu.sync_copy(srci.at[pl.ds(b, CK)], sidx)
            pltpu.sync_copy(dsti.at[pl.ds(b, CK)], didx)
            pltpu.async_copy(_h.at[sidx], rows, sem).wait()
            if _p == 0:
                pltpu.async_copy(sd.at[didx], sdr, sem).wait()
                pltpu.async_copy(ss.at[sidx], ssr, sem).wait()

                def ecomp(k, _):
                    v = sdr[k, :] + ssr[k, :]
                    v = jnp.where(v > 0.0, v, ALPHA * v)
                    ev[k, :] = jnp.exp(-v)
                    return 0
                lax.fori_loop(0, CK, ecomp, 0)
                pltpu.sync_copy(ev, rs.at[didx], add=True)
                pltpu.sync_copy(ev, e8s.at[pl.ds(b, CK)])
            else:
                pltpu.sync_copy(e8s.at[pl.ds(b, CK)], ev)

            def hscale(k, _):
                e8 = ev[k, :]
                e0 = _lane_bcast(e8, _i0)
                e1 = _lane_bcast(e8, _i1)
                for d in range(4):
                    rows[k, pl.ds(d * 16, 16)] = rows[k, pl.ds(d * 16, 16)] * e0
                for d in range(4, 8):
                    rows[k, pl.ds(d * 16, 16)] = rows[k, pl.ds(d * 16, 16)] * e1
                return 0
            lax.fori_loop(0, CK, hscale, 0)
            pltpu.sync_copy(rows, acc.at[didx], add=True)
            return 0
        lax.fori_loop(0, NCHUNK, gat_chunk, 0)
        plsc.subcore_barrier()

        o0, o1 = hpout[p]

        @pl.when(c == 0)
        def _(o=o0):
            pltpu.sync_copy(acc.at[pl.ds(r0, RPT)], o.at[pl.ds(r0, RPT)])

        @pl.when(c == 1)
        def _(o=o1):
            pltpu.sync_copy(acc.at[pl.ds(r0, RPT)], o.at[pl.ds(r0, RPT)])
        plsc.subcore_barrier()

    @pl.when(c == 0)
    def _():
        pltpu.sync_copy(rs.at[pl.ds(r0, RPT)], rs0.at[pl.ds(r0, RPT)])

    @pl.when(c == 1)
    def _():
        pltpu.sync_copy(rs.at[pl.ds(r0, RPT)], rs1.at[pl.ds(r0, RPT)])


def _sc_a(xw1, h0, h1, h2, h3, sd, ss, ew, srci, dsti):
    f32 = jnp.float32
    outs = (
        jax.ShapeDtypeStruct((NPAD, 128), f32),
        jax.ShapeDtypeStruct((NPAD, 128), f32),
        *([jax.ShapeDtypeStruct((NPAD, 128), f32)] * 8),
        jax.ShapeDtypeStruct((NPAD, 16), f32),
        jax.ShapeDtypeStruct((NPAD, 16), f32),
        jax.ShapeDtypeStruct((EPAD, 16), f32),
    )
    scratch = [
        pltpu.VMEM((CK,), jnp.int32),
        pltpu.VMEM((CK,), jnp.int32),
        pltpu.VMEM((CK,), f32),
        pltpu.VMEM((CK, 128), f32),
        pltpu.VMEM((CK, 16), f32),
        pltpu.VMEM((CK, 16), f32),
        pltpu.VMEM((CK, 16), f32),
        pltpu.VMEM((64, 128), f32),
        pltpu.VMEM((64, 16), f32),
        pltpu.VMEM_SHARED((NPAD, 128), f32),
        pltpu.VMEM_SHARED((NPAD, 16), f32),
        pltpu.SemaphoreType.DMA,
    ]
    fn = pl.kernel(_sca_body, out_type=outs, mesh=_mesh(),
                   scratch_types=scratch,
                   compiler_params=pltpu.CompilerParams(use_tc_tiling_on_sc=False))
    return fn(xw1, h0, h1, h2, h3, sd, ss, ew, srci, dsti)


def _tc1_body(g1p0, g1p1, b1r, w2e, hp00, hp01, hp10, hp11, hp20, hp21,
              hp30, hp31, rs0, rs1, outw, a2r,
              yw2_ref, h2_ref, sd2_ref, ss2_ref):
    g1 = g1p0[...] + g1p1[...]
    y1 = jnp.maximum(g1 + b1r[...], 0.0)
    yw2_ref[...] = jnp.dot(y1, w2e[...], preferred_element_type=jnp.float32)
    rs = rs0[...] + rs1[...] + 1e-16
    hps = ((hp00, hp01), (hp10, hp11), (hp20, hp21), (hp30, hp31))
    zparts = []
    for p in range(4):
        hp = hps[p][0][...] + hps[p][1][...]
        for j in range(2):
            hj = hp[:, j * 64:(j + 1) * 64] / rs[:, 2 * p + j:2 * p + j + 1]
            zparts.append(jnp.where(hj > 0.0, hj, jnp.exp(hj) - 1.0))
    z1 = jnp.concatenate(zparts, axis=1)
    h2 = jnp.dot(z1, outw[...], preferred_element_type=jnp.float32)
    h2_ref[...] = h2
    SDSS = jnp.dot(h2, a2r[...], preferred_element_type=jnp.float32)
    sd2_ref[...] = SDSS[:, :16]
    ss2_ref[...] = SDSS[:, 16:]


def _tc1(g1p0, g1p1, b1r, w2e, hps, rs0, rs1, outw, a2r):
    BR = 1280
    grid = NPAD // BR
    f32 = jnp.float32
    row = lambda i: (i, 0)
    full = lambda shape: pl.BlockSpec(shape, lambda i: (0, 0))
    outs = [jax.ShapeDtypeStruct((NPAD, 64), f32),
            jax.ShapeDtypeStruct((NPAD, 64), f32),
            jax.ShapeDtypeStruct((NPAD, 16), f32),
            jax.ShapeDtypeStruct((NPAD, 16), f32)]
    return pl.pallas_call(
        _tc1_body,
        grid=(grid,),
        in_specs=[
            pl.BlockSpec((BR, 128), row), pl.BlockSpec((BR, 128), row),
            full((1, 128)), full((128, 64)),
            *([pl.BlockSpec((BR, 128), row)] * 8),
            pl.BlockSpec((BR, 16), row), pl.BlockSpec((BR, 16), row),
            full((512, 64)), full((64, 32)),
        ],
        out_specs=[
            pl.BlockSpec((BR, 64), row), pl.BlockSpec((BR, 64), row),
            pl.BlockSpec((BR, 16), row), pl.BlockSpec((BR, 16), row),
        ],
        out_shape=outs,
    )(g1p0, g1p1, b1r, w2e, *hps, rs0, rs1, outw, a2r)


def _scb_body(yw2, h2m, sd2, ss2, oh, ew, srci, dsti, idxtr,
              g2p0, g2p1, hq0, hq1, rq0, rq1,
              gtr0, gtr1, htr0, htr1, rtr0, rtr1, ohtr,
              sidx, didx, wv, rowsA, rowsB, sdr, ssr, ev, zb, zb16,
              tidx, trb, trb16, accg, acch, rs2, sem):
    c = lax.axis_index("c")
    s = lax.axis_index("s")
    wid = s * 2 + c
    base0 = wid * EPW
    r0 = s * RPT

    _fill_zero(zb, 64, 64)
    _fill_zero(zb16, 64, 16)
    for i in range(RPT // 64):
        pltpu.sync_copy(zb, accg.at[pl.ds(r0 + i * 64, 64)])
        pltpu.sync_copy(zb, acch.at[pl.ds(r0 + i * 64, 64)])
        pltpu.sync_copy(zb16, rs2.at[pl.ds(r0 + i * 64, 64)])
    plsc.subcore_barrier()

    def chunk(ci, _):
        b = base0 + ci * CK
        pltpu.sync_copy(srci.at[pl.ds(b, CK)], sidx)
        pltpu.sync_copy(dsti.at[pl.ds(b, CK)], didx)
        pltpu.sync_copy(ew.at[pl.ds(b, CK)], wv)
        pltpu.async_copy(yw2.at[sidx], rowsA, sem).wait()
        pltpu.async_copy(h2m.at[sidx], rowsB, sem).wait()
        pltpu.async_copy(sd2.at[didx], sdr, sem).wait()
        pltpu.async_copy(ss2.at[sidx], ssr, sem).wait()

        def one(g, _):
            wvec = wv[pl.ds(g * 16, 16)]
            for j in range(16):
                w = _lane_bcast(wvec, jnp.full((16,), j, jnp.int32))
                k = g * 16 + j
                for d in range(4):
                    rowsA[k, pl.ds(d * 16, 16)] = rowsA[k, pl.ds(d * 16, 16)] * w
                v = sdr[k, :] + ssr[k, :]
                v = jnp.where(v > 0.0, v, ALPHA * v)
                e = jnp.exp(-v)
                ev[k, :] = e
                for d in range(4):
                    rowsB[k, pl.ds(d * 16, 16)] = rowsB[k, pl.ds(d * 16, 16)] * e
            return 0
        lax.fori_loop(0, CK // 16, one, 0)
        pltpu.sync_copy(rowsA, accg.at[didx], add=True)
        pltpu.sync_copy(rowsB, acch.at[didx], add=True)
        pltpu.sync_copy(ev, rs2.at[didx], add=True)
        return 0
    lax.fori_loop(0, NCHUNK, chunk, 0)
    plsc.subcore_barrier()

    @pl.when(c == 0)
    def _():
        pltpu.sync_copy(accg.at[pl.ds(r0, RPT)], g2p0.at[pl.ds(r0, RPT)])
        pltpu.sync_copy(acch.at[pl.ds(r0, RPT)], hq0.at[pl.ds(r0, RPT)])
        pltpu.sync_copy(rs2.at[pl.ds(r0, RPT)], rq0.at[pl.ds(r0, RPT)])

    @pl.when(c == 1)
    def _():
        pltpu.sync_copy(accg.at[pl.ds(r0, RPT)], g2p1.at[pl.ds(r0, RPT)])
        pltpu.sync_copy(acch.at[pl.ds(r0, RPT)], hq1.at[pl.ds(r0, RPT)])
        pltpu.sync_copy(rs2.at[pl.ds(r0, RPT)], rq1.at[pl.ds(r0, RPT)])
    plsc.subcore_barrier()

    t0 = s * TRT
    pltpu.sync_copy(idxtr.at[pl.ds(t0, TRT)], tidx)

    @pl.when(c == 0)
    def _():
        pltpu.async_copy(g2p0.at[tidx], trb, sem).wait()
        pltpu.sync_copy(trb, gtr0.at[pl.ds(t0, TRT)])
        pltpu.async_copy(hq0.at[tidx], trb, sem).wait()
        pltpu.sync_copy(trb, htr0.at[pl.ds(t0, TRT)])
        pltpu.async_copy(rq0.at[tidx], trb16, sem).wait()
        pltpu.sync_copy(trb16, rtr0.at[pl.ds(t0, TRT)])
        pltpu.async_copy(oh.at[tidx], trb, sem).wait()
        pltpu.sync_copy(trb, ohtr.at[pl.ds(t0, TRT)])

    @pl.when(c == 1)
    def _():
        pltpu.async_copy(g2p1.at[tidx], trb, sem).wait()
        pltpu.sync_copy(trb, gtr1.at[pl.ds(t0, TRT)])
        pltpu.async_copy(hq1.at[tidx], trb, sem).wait()
        pltpu.sync_copy(trb, htr1.at[pl.ds(t0, TRT)])
        pltpu.async_copy(rq1.at[tidx], trb16, sem).wait()
        pltpu.sync_copy(trb16, rtr1.at[pl.ds(t0, TRT)])


def _sc_b(yw2, h2m, sd2, ss2, oh, ew, srci, dsti, idxtr):
    f32 = jnp.float32
    outs = (
        jax.ShapeDtypeStruct((NPAD, 64), f32),
        jax.ShapeDtypeStruct((NPAD, 64), f32),
        jax.ShapeDtypeStruct((NPAD, 64), f32),
        jax.ShapeDtypeStruct((NPAD, 64), f32),
        jax.ShapeDtypeStruct((NPAD, 16), f32),
        jax.ShapeDtypeStruct((NPAD, 16), f32),
        jax.ShapeDtypeStruct((NTRP, 64), f32),
        jax.ShapeDtypeStruct((NTRP, 64), f32),
        jax.ShapeDtypeStruct((NTRP, 64), f32),
        jax.ShapeDtypeStruct((NTRP, 64), f32),
        jax.ShapeDtypeStruct((NTRP, 16), f32),
        jax.ShapeDtypeStruct((NTRP, 16), f32),
        jax.ShapeDtypeStruct((NTRP, 64), f32),
    )
    scratch = [
        pltpu.VMEM((CK,), jnp.int32),
        pltpu.VMEM((CK,), jnp.int32),
        pltpu.VMEM((CK,), f32),
        pltpu.VMEM((CK, 64), f32),
        pltpu.VMEM((CK, 64), f32),
        pltpu.VMEM((CK, 16), f32),
        pltpu.VMEM((CK, 16), f32),
        pltpu.VMEM((CK, 16), f32),
        pltpu.VMEM((64, 64), f32),
        pltpu.VMEM((64, 16), f32),
        pltpu.VMEM((TRT,), jnp.int32),
        pltpu.VMEM((TRT, 64), f32),
        pltpu.VMEM((TRT, 16), f32),
        pltpu.VMEM_SHARED((NPAD, 64), f32),
        pltpu.VMEM_SHARED((NPAD, 64), f32),
        pltpu.VMEM_SHARED((NPAD, 16), f32),
        pltpu.SemaphoreType.DMA,
    ]
    fn = pl.kernel(_scb_body, out_type=outs, mesh=_mesh(),
                   scratch_types=scratch,
                   compiler_params=pltpu.CompilerParams(use_tc_tiling_on_sc=False))
    return fn(yw2, h2m, sd2, ss2, oh, ew, srci, dsti, idxtr)


def _tc2_body(g2p0, g2p1, b2r, hq0, hq1, rq0, rq1,
              gtr0, gtr1, htr0, htr1, rtr0, rtr1, ohtr,
              y_ref, z_ref, loss_ref, accs):
    pid = pl.program_id(0)
    y = g2p0[...] + g2p1[...] + b2r[...]
    y_ref[...] = y
    rs = rq0[...] + rq1[...] + 1e-16
    zp = (hq0[...] + hq1[...]) / rs[:, 0:1]
    z = jnp.where(zp > 0.0, zp, jnp.exp(zp) - 1.0)
    z_ref[...] = z
    BR = y.shape[0]
    grow = pid * BR + lax.broadcasted_iota(jnp.int32, (BR, 1), 0)
    rmask = (grow < N).astype(jnp.float32)
    ym = jnp.max(y, axis=1, keepdims=True)
    ly = y - ym
    ly = ly - jnp.log(jnp.sum(jnp.exp(ly), axis=1, keepdims=True))
    ze = jnp.exp(z - jnp.max(z, axis=1, keepdims=True))
    sz = ze / jnp.sum(ze, axis=1, keepdims=True)
    cl_part = jnp.sum(rmask * sz * ly)

    @pl.when(pid == 0)
    def _():
        ytr = gtr0[...] + gtr1[...] + b2r[...]
        rstr = rtr0[...] + rtr1[...] + 1e-16
        zpre = (htr0[...] + htr1[...]) / rstr[:, 0:1]
        ztr = jnp.where(zpre > 0.0, zpre, jnp.exp(zpre) - 1.0)
        oh = ohtr[...]
        tmask = (lax.broadcasted_iota(jnp.int32, (NTRP, 1), 0) < NTR
                 ).astype(jnp.float32)
        ymx = jnp.max(ytr, axis=1, keepdims=True)
        lse_y = ymx + jnp.log(jnp.sum(jnp.exp(ytr - ymx), axis=1, keepdims=True))
        zmx = jnp.max(ztr, axis=1, keepdims=True)
        lse_z = zmx + jnp.log(jnp.sum(jnp.exp(ztr - zmx), axis=1, keepdims=True))
        pick_y = jnp.sum(oh * ytr, axis=1, keepdims=True)
        pick_z = jnp.sum(oh * ztr, axis=1, keepdims=True)
        ce1 = jnp.sum(tmask * (lse_y - pick_y)) / NTR
        ce2 = jnp.sum(tmask * (lse_z - pick_z)) / NTR
        io0 = lax.broadcasted_iota(jnp.int32, (1, 128), 1)
        accs[...] = jnp.where(io0 == 0, ce1, 0.0) + jnp.where(io0 == 1, ce2, 0.0)

    io = lax.broadcasted_iota(jnp.int32, (1, 128), 1)
    accs[...] = accs[...] + jnp.where(io == 2, cl_part, 0.0)
    scale = jnp.where(io == 2, -1.0 / N, 1.0)
    loss_ref[...] = accs[...] * scale


def _tc2(g2p0, g2p1, b2r, hq0, hq1, rq0, rq1,
         gtr0, gtr1, htr0, htr1, rtr0, rtr1, ohtr):
    f32 = jnp.float32
    BR = 1280
    grid = NPAD // BR
    row = lambda i: (i, 0)
    full = lambda shape: pl.BlockSpec(shape, lambda i: (0, 0))
    outs = [jax.ShapeDtypeStruct((NPAD, 64), f32),
            jax.ShapeDtypeStruct((NPAD, 64), f32),
            jax.ShapeDtypeStruct((1, 128), f32)]
    return pl.pallas_call(
        _tc2_body,
        grid=(grid,),
        in_specs=[
            pl.BlockSpec((BR, 64), row), pl.BlockSpec((BR, 64), row),
            full((1, 64)),
            pl.BlockSpec((BR, 64), row), pl.BlockSpec((BR, 64), row),
            pl.BlockSpec((BR, 16), row), pl.BlockSpec((BR, 16), row),
            full((NTRP, 64)), full((NTRP, 64)),
            full((NTRP, 64)), full((NTRP, 64)),
            full((NTRP, 16)), full((NTRP, 16)),
            full((NTRP, 64)),
        ],
        out_specs=[
            pl.BlockSpec((BR, 64), row), pl.BlockSpec((BR, 64), row),
            full((1, 128)),
        ],
        out_shape=outs,
        scratch_shapes=[pltpu.VMEM((1, 128), f32)],
    )(g2p0, g2p1, b2r, hq0, hq1, rq0, rq1,
      gtr0, gtr1, htr0, htr1, rtr0, rtr1, ohtr)


def kernel(x, edge_weight, gc1_W, gc1_b, gc2_W, gc2_b, att_W, att_a,
              out_W, out_a, edge_index, idx_train, labels):
    f32 = jnp.float32
    dst = edge_index[0]
    src = edge_index[1]
    pe = EPAD - E
    dst_p = jnp.concatenate([dst, jnp.full((pe,), N, jnp.int32)])
    src_p = jnp.concatenate([src, jnp.full((pe,), N, jnp.int32)])
    ew_p = jnp.concatenate([edge_weight, jnp.zeros((pe,), f32)])
    x_p = jnp.pad(x, ((0, NPAD - N), (0, 0)))
    lab_p = jnp.pad(labels, (0, NPAD - N)).reshape(NPAD, 1)
    idxtr_p = jnp.pad(idx_train, (0, NTRP - NTR))
    W_all = att_W.transpose(1, 0, 2).reshape(NFEAT, NHEADS * NHID2)
    wbig = jnp.concatenate([gc1_W, W_all], axis=1)
    eye8 = jnp.eye(NHEADS, dtype=f32)
    A_sd = (att_a[:, 0, :NHID2][:, :, None] * eye8[:, None, :]).reshape(512, 8)
    A_ss = (att_a[:, 0, NHID2:][:, :, None] * eye8[:, None, :]).reshape(512, 8)
    zpad8 = jnp.zeros((512, 8), f32)
    asdss = jnp.concatenate(
        [A_sd, zpad8, A_ss, zpad8], axis=1)
    w2e = gc2_W[:NHID1] + gc2_W[NHID1:2 * NHID1] + gc2_W[2 * NHID1:]
    b1r = gc1_b.reshape(1, NHID1)
    b2r = gc2_b.reshape(1, NCLASS)
    a2r = jnp.concatenate(
        [jnp.tile(out_a[0, :NCLASS][:, None], (1, 16)),
         jnp.tile(out_a[0, NCLASS:][:, None], (1, 16))], axis=1)

    xw1, h0, h1, h2, h3, sdv, ssv, oh = _tc0(x_p, wbig, asdss, lab_p)
    (g1p0, g1p1, hp00, hp01, hp10, hp11, hp20, hp21, hp30, hp31,
     rs0, rs1, _e8) = _sc_a(xw1, h0, h1, h2, h3, sdv, ssv, ew_p, src_p, dst_p)
    yw2, h2m, sd2, ss2 = _tc1(
        g1p0, g1p1, b1r, w2e,
        (hp00, hp01, hp10, hp11, hp20, hp21, hp30, hp31),
        rs0, rs1, out_W, a2r)
    (g2p0, g2p1, hq0, hq1, rq0, rq1, gtr0, gtr1, htr0, htr1,
     rtr0, rtr1, ohtr) = _sc_b(yw2, h2m, sd2, ss2, oh, ew_p, src_p, dst_p,
                               idxtr_p)
    yf, zf, L = _tc2(g2p0, g2p1, b2r, hq0, hq1, rq0, rq1,
                     gtr0, gtr1, htr0, htr1, rtr0, rtr1, ohtr)
    y = yf[:N]
    z = zf[:N]
    ce1 = L[0, 0]
    ce2 = L[0, 1]
    cl = L[0, 2]
    return (y, z, ce1, ce2, ce1 + 0.05 * cl, ce2 + 0.05 * cl)
